# baseline (speedup 1.0000x reference)
import functools

import jax
import jax.numpy as jnp
from jax import lax
from jax.experimental import pallas as pl
from jax.experimental.pallas import tpu as pltpu
from jax.experimental.pallas import tpu_sc as plsc

N = 50000
D = 128
NP = 50176
R = 784
NPASS = 2
CH = 2048
SENT = 3.0e38
BLK = 1024


def _leaky(x):
    return jnp.where(x >= 0, x, 0.01 * x)


def _enc_body(x_ref, w_ref, b_ref, o_ref):
    y = lax.dot(x_ref[0], w_ref[0], preferred_element_type=jnp.float32)
    o_ref[0] = _leaky(y + b_ref[0])


def _encode(x3, w3, b3):
    return pl.pallas_call(
        _enc_body,
        grid=(3, NP // BLK),
        in_specs=[
            pl.BlockSpec((1, BLK, 128), lambda i, j: (i, j, 0)),
            pl.BlockSpec((1, 128, 128), lambda i, j: (i, 0, 0)),
            pl.BlockSpec((1, 1, 128), lambda i, j: (i, 0, 0)),
        ],
        out_specs=pl.BlockSpec((1, BLK, 128), lambda i, j: (i, j, 0)),
        out_shape=jax.ShapeDtypeStruct((3, NP, 128), jnp.float32),
    )(x3, w3, b3)


def _dense_body(xd_ref, mn_ref, w1_ref, w2_ref, b_ref, o_ref):
    xd = xd_ref[...]
    mn = mn_ref[...]
    mx = jnp.where(mn > 1e30, 0.0, xd - mn)
    h = (lax.dot(xd, w1_ref[...], preferred_element_type=jnp.float32)
         + lax.dot(mx, w2_ref[...], preferred_element_type=jnp.float32)
         + b_ref[...])
    o_ref[...] = xd + _leaky(h)


def _dense(xd, mn, w, b):
    return pl.pallas_call(
        _dense_body,
        grid=(NP // BLK,),
        in_specs=[
            pl.BlockSpec((BLK, 128), lambda i: (i, 0)),
            pl.BlockSpec((BLK, 128), lambda i: (i, 0)),
            pl.BlockSpec((128, 128), lambda i: (0, 0)),
            pl.BlockSpec((128, 128), lambda i: (0, 0)),
            pl.BlockSpec((1, 128), lambda i: (0, 0)),
        ],
        out_specs=pl.BlockSpec((BLK, 128), lambda i: (i, 0)),
        out_shape=jax.ShapeDtypeStruct((NP, 128), jnp.float32),
    )(xd, mn, w[:D], w[D:], b[None])


def _make_segmin(e_pad):
    nchunks = e_pad // CH
    assert nchunks % 2 == 0
    mesh = plsc.VectorSubcoreMesh(core_axis_name="c", subcore_axis_name="s")

    @functools.partial(
        pl.kernel,
        out_type=jax.ShapeDtypeStruct((NP, 128), jnp.float32),
        mesh=mesh,
        scratch_types=[
            pltpu.VMEM((R, 128), jnp.float32),
            pltpu.VMEM((CH,), jnp.int32),
            pltpu.VMEM((CH,), jnp.int32),
            pltpu.VMEM((CH,), jnp.int32),
            pltpu.VMEM((CH,), jnp.int32),
            pltpu.VMEM((32,), jnp.int32),
            pltpu.VMEM((32,), jnp.int32),
            pltpu.VMEM((16, 128), jnp.float32),
            pltpu.SemaphoreType.DMA,
            pltpu.SemaphoreType.DMA,
            pltpu.SemaphoreType.DMA,
        ],
        compiler_params=pltpu.CompilerParams(needs_layout_passes=False),
    )
    def segmin(xsrc, e0h, e1h, out, acc, e0a, e1a, e0b, e1b, psrc, pdl, rows,
               sem, sema, semb):
        wid = lax.axis_index("s") * 2 + lax.axis_index("c")
        iota = lax.iota(jnp.int32, 16)
        cols = [(k * 16 + iota) for k in range(8)]
        big = jnp.int32(1 << 20)

        def do_flush(cnt):
            dlv = pdl[pl.ds(0, 16)]
            srcs = psrc[pl.ds(0, 16)]
            valid = iota < cnt
            skey, ssrc = plsc.sort_key_val(jnp.where(valid, dlv, big), srcs)
            pltpu.async_copy(xsrc.at[ssrc], rows, sem).wait()
            dls = [jnp.sum(jnp.where(iota == r, skey, 0)) for r in range(16)]
            dupf = [(dls[r] == dls[r - 1]) & (dls[r] < big) for r in range(1, 16)]
            anydup = sum(f.astype(jnp.int32) for f in dupf)

            @pl.when(anydup > 0)
            def _():
                for r in range(1, 16):
                    dm = jnp.broadcast_to(dupf[r - 1], (16,))
                    for k in range(8):
                        a = rows[r - 1, pl.ds(k * 16, 16)]
                        b = rows[r, pl.ds(k * 16, 16)]
                        rows[r, pl.ds(k * 16, 16)] = jnp.where(
                            dm, jnp.minimum(a, b), b)

            for r in range(16):
                if r < 15:
                    okr = (dls[r] != dls[r + 1]) & (dls[r] < big)
                else:
                    okr = dls[r] < big
                okv = jnp.broadcast_to(okr, (16,))
                dlr = jnp.broadcast_to(jnp.minimum(dls[r], R - 1), (16,))
                for k in range(8):
                    a = plsc.load_gather(acc, [dlr, cols[k]])
                    v = jnp.minimum(a, rows[r, pl.ds(k * 16, 16)])
                    plsc.store_scatter(acc, [dlr, cols[k]], v, mask=okv)

            tcnt = jnp.maximum(cnt - 16, 0)
            s2 = psrc[pl.ds(16, 16)]
            d2 = pdl[pl.ds(16, 16)]
            tm = iota < tcnt
            plsc.store_scatter(psrc, [iota], s2, mask=tm)
            plsc.store_scatter(pdl, [iota], d2, mask=tm)
            return tcnt

        def scan_chunk(e0v, e1v, lo, cnt):
            def vec_body(j, cnt):
                d = e1v[pl.ds(j * 16, 16)]
                m = (d >= lo) & (d < lo + R)

                def hit(cnt):
                    s = e0v[pl.ds(j * 16, 16)]
                    mi = m.astype(jnp.int32)
                    nh = jnp.sum(mi)
                    pos = cnt + plsc.cumsum(mi) - mi
                    plsc.store_scatter(psrc, [pos], s, mask=m)
                    plsc.store_scatter(pdl, [pos], d - lo, mask=m)
                    return lax.while_loop(lambda c: c >= 16, do_flush,
                                          cnt + nh)
                return lax.cond(jnp.any(m), hit, lambda c: c, cnt)
            return lax.fori_loop(0, CH // 16, vec_body, cnt)

        def issue(ci, e0v, e1v, s):
            pltpu.async_copy(e0h.at[pl.ds(ci * CH, CH)], e0v, s)
            pltpu.async_copy(e1h.at[pl.ds(ci * CH, CH)], e1v, s)

        def drain(ci, e0v, e1v, s):
            pltpu.make_async_copy(e0h.at[pl.ds(ci * CH, CH)], e0v, s).wait()
            pltpu.make_async_copy(e1h.at[pl.ds(ci * CH, CH)], e1v, s).wait()

        def pass_body(p, _):
            lo = (p * 32 + wid) * R

            def initbody(i, _):
                for k in range(8):
                    acc[i, pl.ds(k * 16, 16)] = jnp.full((16,), SENT,
                                                         jnp.float32)
                return 0
            lax.fori_loop(0, R, initbody, 0)
            zeros = jnp.zeros((16,), jnp.int32)
            psrc[pl.ds(0, 16)] = zeros
            psrc[pl.ds(16, 16)] = zeros
            pdl[pl.ds(0, 16)] = zeros
            pdl[pl.ds(16, 16)] = zeros

            issue(jnp.int32(0), e0a, e1a, sema)
            issue(jnp.int32(1), e0b, e1b, semb)

            def pair_body(pi, cnt):
                ca = pi * 2
                drain(ca, e0a, e1a, sema)
                cnt = scan_chunk(e0a, e1a, lo, cnt)
                issue(ca + 2, e0a, e1a, sema)
                drain(ca + 1, e0b, e1b, semb)
                cnt = scan_chunk(e0b, e1b, lo, cnt)
                issue(ca + 3, e0b, e1b, semb)
                return cnt

            cnt = lax.fori_loop(0, nchunks // 2, pair_body, jnp.int32(0))
            drain(jnp.int32(nchunks), e0a, e1a, sema)
            drain(jnp.int32(nchunks + 1), e0b, e1b, semb)
            cnt = lax.while_loop(lambda c: c > 0, do_flush, cnt)
            pltpu.sync_copy(acc, out.at[pl.ds(lo, R)])
            return 0

        lax.fori_loop(0, NPASS, pass_body, 0)

    return segmin


def _prep_edges(e_src, e_dst):
    e = e_src.shape[0]
    ep = ((e + 2 * CH - 1) // (2 * CH)) * (2 * CH)
    s = jnp.pad(e_src, (0, ep + 2 * CH - e))
    d = jnp.pad(e_dst, (0, ep + 2 * CH - e), constant_values=1 << 20)
    return s, d, ep


def kernel(vertices, edges, faces, edge_to_vertex, face_to_edge, face_to_face,
           W_v, b_v, W_e, b_e, W_f, b_f, W_v2e, b_v2e, W_e2f, b_e2f,
           W_l0, b_l0, W_l1, b_l1, W_l2, b_l2):
    def padfeat(x, w):
        xp = jnp.pad(x, ((0, NP - N), (0, 128 - x.shape[1])))
        wp = jnp.pad(w, ((0, 128 - w.shape[0]), (0, 0)))
        return xp, wp

    xv_p, wv_p = padfeat(vertices, W_v)
    xe_p, we_p = padfeat(edges, W_e)
    xf_p, wf_p = padfeat(faces, W_f)
    enc = _encode(jnp.stack([xv_p, xe_p, xf_p]),
                  jnp.stack([wv_p, we_p, wf_p]),
                  jnp.stack([b_v, b_e, b_f])[:, None, :])
    x_v, x_e, x_f = enc[0], enc[1], enc[2]

    ev_s, ev_d, ev_n = _prep_edges(edge_to_vertex[1], edge_to_vertex[0])
    fe_s, fe_d, fe_n = _prep_edges(face_to_edge[1], face_to_edge[0])
    ff_s, ff_d, ff_n = _prep_edges(face_to_face[0], face_to_face[1])

    mn = _make_segmin(ev_n)(x_v, ev_s, ev_d)
    x_e = _dense(x_e, mn, W_v2e, b_v2e)
    mn = _make_segmin(fe_n)(x_e, fe_s, fe_d)
    x_f = _dense(x_f, mn, W_e2f, b_e2f)
    segff = _make_segmin(ff_n)
    for w, b in ((W_l0, b_l0), (W_l1, b_l1), (W_l2, b_l2)):
        mn = segff(x_f, ff_s, ff_d)
        x_f = _dense(x_f, mn, w, b)
    return x_f[:N]


# speedup vs baseline: 1.5002x; 1.5002x over previous
import functools

import jax
import jax.numpy as jnp
from jax import lax
from jax.experimental import pallas as pl
from jax.experimental.pallas import tpu as pltpu
from jax.experimental.pallas import tpu_sc as plsc

N = 50000
D = 128
NP = 50176
R = 784
NPASS = 2
CH = 2048
SENT = 3.0e38
BLK = 1024


def _leaky(x):
    return jnp.where(x >= 0, x, 0.01 * x)


def _enc_body(x_ref, w_ref, b_ref, o_ref):
    y = lax.dot(x_ref[0], w_ref[0], preferred_element_type=jnp.float32)
    o_ref[0] = _leaky(y + b_ref[0])


def _encode(x3, w3, b3):
    return pl.pallas_call(
        _enc_body,
        grid=(3, NP // BLK),
        in_specs=[
            pl.BlockSpec((1, BLK, 128), lambda i, j: (i, j, 0)),
            pl.BlockSpec((1, 128, 128), lambda i, j: (i, 0, 0)),
            pl.BlockSpec((1, 1, 128), lambda i, j: (i, 0, 0)),
        ],
        out_specs=pl.BlockSpec((1, BLK, 128), lambda i, j: (i, j, 0)),
        out_shape=jax.ShapeDtypeStruct((3, NP, 128), jnp.float32),
    )(x3, w3, b3)


def _dense_body(xd_ref, mn_ref, w1_ref, w2_ref, b_ref, o_ref):
    xd = xd_ref[...]
    mn = mn_ref[...]
    mx = jnp.where(mn > 1e30, 0.0, xd - mn)
    h = (lax.dot(xd, w1_ref[...], preferred_element_type=jnp.float32)
         + lax.dot(mx, w2_ref[...], preferred_element_type=jnp.float32)
         + b_ref[...])
    o_ref[...] = xd + _leaky(h)


def _dense(xd, mn, w, b):
    return pl.pallas_call(
        _dense_body,
        grid=(NP // BLK,),
        in_specs=[
            pl.BlockSpec((BLK, 128), lambda i: (i, 0)),
            pl.BlockSpec((BLK, 128), lambda i: (i, 0)),
            pl.BlockSpec((128, 128), lambda i: (0, 0)),
            pl.BlockSpec((128, 128), lambda i: (0, 0)),
            pl.BlockSpec((1, 128), lambda i: (0, 0)),
        ],
        out_specs=pl.BlockSpec((BLK, 128), lambda i: (i, 0)),
        out_shape=jax.ShapeDtypeStruct((NP, 128), jnp.float32),
    )(xd, mn, w[:D], w[D:], b[None])


def _make_segmin(e_pad):
    nchunks = e_pad // CH
    assert nchunks % 2 == 0
    mesh = plsc.VectorSubcoreMesh(core_axis_name="c", subcore_axis_name="s")

    @functools.partial(
        pl.kernel,
        out_type=jax.ShapeDtypeStruct((NP, 128), jnp.float32),
        mesh=mesh,
        scratch_types=[
            pltpu.VMEM((R, 128), jnp.float32),
            pltpu.VMEM((CH,), jnp.int32),
            pltpu.VMEM((CH,), jnp.int32),
            pltpu.VMEM((CH,), jnp.int32),
            pltpu.VMEM((CH,), jnp.int32),
            pltpu.VMEM((2080,), jnp.int32),
            pltpu.VMEM((2080,), jnp.int32),
            pltpu.VMEM((16, 128), jnp.float32),
            pltpu.SemaphoreType.DMA,
            pltpu.SemaphoreType.DMA,
            pltpu.SemaphoreType.DMA,
        ],
        compiler_params=pltpu.CompilerParams(needs_layout_passes=False),
    )
    def segmin(xsrc, e0h, e1h, out, acc, e0a, e1a, e0b, e1b, psrc, pdl, rows,
               sem, sema, semb):
        wid = lax.axis_index("s") * 2 + lax.axis_index("c")
        iota = lax.iota(jnp.int32, 16)
        cols = [(k * 16 + iota) for k in range(8)]
        big = jnp.int32(1 << 20)

        def do_rmw(skey):
            dls = [jnp.sum(jnp.where(iota == r, skey, 0)) for r in range(16)]
            dupf = [(dls[r] == dls[r - 1]) & (dls[r] < big)
                    for r in range(1, 16)]
            anydup = sum(f.astype(jnp.int32) for f in dupf)

            @pl.when(anydup > 0)
            def _():
                for r in range(1, 16):
                    dm = jnp.broadcast_to(dupf[r - 1], (16,))
                    for k in range(8):
                        a = rows[r - 1, pl.ds(k * 16, 16)]
                        b = rows[r, pl.ds(k * 16, 16)]
                        rows[r, pl.ds(k * 16, 16)] = jnp.where(
                            dm, jnp.minimum(a, b), b)

            for r in range(16):
                if r < 15:
                    okr = (dls[r] != dls[r + 1]) & (dls[r] < big)
                else:
                    okr = dls[r] < big
                okv = jnp.broadcast_to(okr, (16,))
                dlr = jnp.broadcast_to(jnp.minimum(dls[r], R - 1), (16,))
                for k in range(8):
                    a = plsc.load_gather(acc, [dlr, cols[k]])
                    v = jnp.minimum(a, rows[r, pl.ds(k * 16, 16)])
                    plsc.store_scatter(acc, [dlr, cols[k]], v, mask=okv)

        def gather_wait():
            pltpu.make_async_copy(xsrc.at[iota], rows, sem).wait()

        def do_flush(t):
            cnt, infl, gsk = t
            dlv = pdl[pl.ds(0, 16)]
            srcs = psrc[pl.ds(0, 16)]
            valid = iota < cnt
            skey, ssrc = plsc.sort_key_val(jnp.where(valid, dlv, big), srcs)

            @pl.when(infl > 0)
            def _():
                gather_wait()
                do_rmw(gsk)

            pltpu.async_copy(xsrc.at[ssrc], rows, sem)
            tcnt = jnp.maximum(cnt - 16, 0)
            s2 = psrc[pl.ds(16, 16)]
            d2 = pdl[pl.ds(16, 16)]
            tm = iota < tcnt
            plsc.store_scatter(psrc, [iota], s2, mask=tm)
            plsc.store_scatter(pdl, [iota], d2, mask=tm)
            return tcnt, jnp.int32(1), skey

        def scan_chunk(e0v, e1v, lo, t):
            def vec_body(j, t):
                d = e1v[pl.ds(j * 16, 16)]
                m = (d >= lo) & (d < lo + R)
                mi = m.astype(jnp.int32)
                nh = jnp.sum(mi)

                def hit(t):
                    cnt, infl, gsk = t
                    s = e0v[pl.ds(j * 16, 16)]
                    pos = cnt + plsc.cumsum(mi) - mi
                    plsc.store_scatter(psrc, [pos], s, mask=m)
                    plsc.store_scatter(pdl, [pos], d - lo, mask=m)
                    return lax.while_loop(lambda u: u[0] >= 16, do_flush,
                                          (cnt + nh, infl, gsk))
                return lax.cond(nh > 0, hit, lambda u: u, t)
            return lax.fori_loop(0, CH // 16, vec_body, t)

        def issue(ci, e0v, e1v, s):
            pltpu.async_copy(e0h.at[pl.ds(ci * CH, CH)], e0v, s)
            pltpu.async_copy(e1h.at[pl.ds(ci * CH, CH)], e1v, s)

        def drain(ci, e0v, e1v, s):
            pltpu.make_async_copy(e0h.at[pl.ds(ci * CH, CH)], e0v, s).wait()
            pltpu.make_async_copy(e1h.at[pl.ds(ci * CH, CH)], e1v, s).wait()

        def pass_body(p, _):
            lo = (p * 32 + wid) * R

            def initbody(i, _):
                for k in range(8):
                    acc[i, pl.ds(k * 16, 16)] = jnp.full((16,), SENT,
                                                         jnp.float32)
                return 0
            lax.fori_loop(0, R, initbody, 0)
            zeros = jnp.zeros((16,), jnp.int32)
            psrc[pl.ds(0, 16)] = zeros
            psrc[pl.ds(16, 16)] = zeros
            pdl[pl.ds(0, 16)] = zeros
            pdl[pl.ds(16, 16)] = zeros

            issue(jnp.int32(0), e0a, e1a, sema)
            issue(jnp.int32(1), e0b, e1b, semb)
            t = (jnp.int32(0), jnp.int32(0), jnp.zeros((16,), jnp.int32))

            def pair_body(pi, t):
                ca = pi * 2
                drain(ca, e0a, e1a, sema)
                t = scan_chunk(e0a, e1a, lo, t)
                issue(ca + 2, e0a, e1a, sema)
                drain(ca + 1, e0b, e1b, semb)
                t = scan_chunk(e0b, e1b, lo, t)
                issue(ca + 3, e0b, e1b, semb)
                return t

            t = lax.fori_loop(0, nchunks // 2, pair_body, t)
            drain(jnp.int32(nchunks), e0a, e1a, sema)
            drain(jnp.int32(nchunks + 1), e0b, e1b, semb)
            t = lax.while_loop(lambda u: u[0] > 0, do_flush, t)

            @pl.when(t[1] > 0)
            def _():
                gather_wait()
                do_rmw(t[2])

            pltpu.sync_copy(acc, out.at[pl.ds(lo, R)])
            return 0

        lax.fori_loop(0, NPASS, pass_body, 0)

    return segmin


def _prep_edges(e_src, e_dst):
    e = e_src.shape[0]
    ep = ((e + 2 * CH - 1) // (2 * CH)) * (2 * CH)
    s = jnp.pad(e_src, (0, ep + 2 * CH - e))
    d = jnp.pad(e_dst, (0, ep + 2 * CH - e), constant_values=1 << 20)
    return s, d, ep


def kernel(vertices, edges, faces, edge_to_vertex, face_to_edge, face_to_face,
           W_v, b_v, W_e, b_e, W_f, b_f, W_v2e, b_v2e, W_e2f, b_e2f,
           W_l0, b_l0, W_l1, b_l1, W_l2, b_l2):
    def padfeat(x, w):
        xp = jnp.pad(x, ((0, NP - N), (0, 128 - x.shape[1])))
        wp = jnp.pad(w, ((0, 128 - w.shape[0]), (0, 0)))
        return xp, wp

    xv_p, wv_p = padfeat(vertices, W_v)
    xe_p, we_p = padfeat(edges, W_e)
    xf_p, wf_p = padfeat(faces, W_f)
    enc = _encode(jnp.stack([xv_p, xe_p, xf_p]),
                  jnp.stack([wv_p, we_p, wf_p]),
                  jnp.stack([b_v, b_e, b_f])[:, None, :])
    x_v, x_e, x_f = enc[0], enc[1], enc[2]

    ev_s, ev_d, ev_n = _prep_edges(edge_to_vertex[1], edge_to_vertex[0])
    fe_s, fe_d, fe_n = _prep_edges(face_to_edge[1], face_to_edge[0])
    ff_s, ff_d, ff_n = _prep_edges(face_to_face[0], face_to_face[1])

    mn = _make_segmin(ev_n)(x_v, ev_s, ev_d)
    x_e = _dense(x_e, mn, W_v2e, b_v2e)
    mn = _make_segmin(fe_n)(x_e, fe_s, fe_d)
    x_f = _dense(x_f, mn, W_e2f, b_e2f)
    segff = _make_segmin(ff_n)
    for w, b in ((W_l0, b_l0), (W_l1, b_l1), (W_l2, b_l2)):
        mn = segff(x_f, ff_s, ff_d)
        x_f = _dense(x_f, mn, w, b)
    return x_f[:N]


# speedup vs baseline: 1.9924x; 1.3281x over previous
import functools

import jax
import jax.numpy as jnp
from jax import lax
from jax.experimental import pallas as pl
from jax.experimental.pallas import tpu as pltpu
from jax.experimental.pallas import tpu_sc as plsc

N = 50000
D = 128
NP = 50176
R = 784
NPASS = 2
CH = 2048
SENT = 3.0e38
BLK = 1024


def _leaky(x):
    return jnp.where(x >= 0, x, 0.01 * x)


def _enc_body(x_ref, w_ref, b_ref, o_ref):
    y = lax.dot(x_ref[0], w_ref[0], preferred_element_type=jnp.float32)
    o_ref[0] = _leaky(y + b_ref[0])


def _encode(x3, w3, b3):
    return pl.pallas_call(
        _enc_body,
        grid=(3, NP // BLK),
        in_specs=[
            pl.BlockSpec((1, BLK, 128), lambda i, j: (i, j, 0)),
            pl.BlockSpec((1, 128, 128), lambda i, j: (i, 0, 0)),
            pl.BlockSpec((1, 1, 128), lambda i, j: (i, 0, 0)),
        ],
        out_specs=pl.BlockSpec((1, BLK, 128), lambda i, j: (i, j, 0)),
        out_shape=jax.ShapeDtypeStruct((3, NP, 128), jnp.float32),
    )(x3, w3, b3)


def _dense_body(xd_ref, mn_ref, w1_ref, w2_ref, b_ref, o_ref):
    xd = xd_ref[...]
    mn = mn_ref[...]
    mx = jnp.where(mn > 1e30, 0.0, xd - mn)
    h = (lax.dot(xd, w1_ref[...], preferred_element_type=jnp.float32)
         + lax.dot(mx, w2_ref[...], preferred_element_type=jnp.float32)
         + b_ref[...])
    o_ref[...] = xd + _leaky(h)


def _dense(xd, mn, w, b):
    return pl.pallas_call(
        _dense_body,
        grid=(NP // BLK,),
        in_specs=[
            pl.BlockSpec((BLK, 128), lambda i: (i, 0)),
            pl.BlockSpec((BLK, 128), lambda i: (i, 0)),
            pl.BlockSpec((128, 128), lambda i: (0, 0)),
            pl.BlockSpec((128, 128), lambda i: (0, 0)),
            pl.BlockSpec((1, 128), lambda i: (0, 0)),
        ],
        out_specs=pl.BlockSpec((BLK, 128), lambda i: (i, 0)),
        out_shape=jax.ShapeDtypeStruct((NP, 128), jnp.float32),
    )(xd, mn, w[:D], w[D:], b[None])


def _make_segmin(e_pad):
    nchunks = e_pad // CH
    assert nchunks % 2 == 0
    mesh = plsc.VectorSubcoreMesh(core_axis_name="c", subcore_axis_name="s")

    @functools.partial(
        pl.kernel,
        out_type=jax.ShapeDtypeStruct((NP, 128), jnp.float32),
        mesh=mesh,
        scratch_types=[
            pltpu.VMEM((R, 128), jnp.float32),
            pltpu.VMEM((CH,), jnp.int32),
            pltpu.VMEM((CH,), jnp.int32),
            pltpu.VMEM((CH,), jnp.int32),
            pltpu.VMEM((CH,), jnp.int32),
            pltpu.VMEM((2080,), jnp.int32),
            pltpu.VMEM((2080,), jnp.int32),
            pltpu.VMEM((16, 128), jnp.float32),
            pltpu.SemaphoreType.DMA,
            pltpu.SemaphoreType.DMA,
            pltpu.SemaphoreType.DMA,
        ],
        compiler_params=pltpu.CompilerParams(needs_layout_passes=False),
    )
    def segmin(xsrc, e0h, e1h, out, acc, e0a, e1a, e0b, e1b, psrc, pdl, rows,
               sem, sema, semb):
        wid = lax.axis_index("s") * 2 + lax.axis_index("c")
        iota = lax.iota(jnp.int32, 16)
        cols = [(k * 16 + iota) for k in range(8)]
        big = jnp.int32(1 << 20)

        def do_rmw(skey):
            dls = [jnp.sum(jnp.where(iota == r, skey, 0)) for r in range(16)]
            dupf = [(dls[r] == dls[r - 1]) & (dls[r] < big)
                    for r in range(1, 16)]
            anydup = sum(f.astype(jnp.int32) for f in dupf)

            @pl.when(anydup > 0)
            def _():
                for r in range(1, 16):
                    dm = jnp.broadcast_to(dupf[r - 1], (16,))
                    for k in range(8):
                        a = rows[r - 1, pl.ds(k * 16, 16)]
                        b = rows[r, pl.ds(k * 16, 16)]
                        rows[r, pl.ds(k * 16, 16)] = jnp.where(
                            dm, jnp.minimum(a, b), b)

            for r in range(16):
                if r < 15:
                    okr = (dls[r] != dls[r + 1]) & (dls[r] < big)
                else:
                    okr = dls[r] < big
                okv = jnp.broadcast_to(okr, (16,))
                dlr = jnp.broadcast_to(jnp.minimum(dls[r], R - 1), (16,))
                for k in range(8):
                    a = plsc.load_gather(acc, [dlr, cols[k]])
                    v = jnp.minimum(a, rows[r, pl.ds(k * 16, 16)])
                    plsc.store_scatter(acc, [dlr, cols[k]], v, mask=okv)

        def gather_wait():
            pltpu.make_async_copy(xsrc.at[iota], rows, sem).wait()

        def flush_any(dlv, srcs, infl, gsk):
            skey, ssrc = plsc.sort_key_val(dlv, srcs)

            @pl.when(infl > 0)
            def _():
                gather_wait()
                do_rmw(gsk)

            pltpu.async_copy(xsrc.at[ssrc], rows, sem)
            return jnp.int32(1), skey

        def scan_chunk(e0v, e1v, lo, cntv):
            def vec_body(j, cntv):
                d = e1v[pl.ds(j * 16, 16)]
                m = (d >= lo) & (d < lo + R)
                mi = m.astype(jnp.int32)
                s = e0v[pl.ds(j * 16, 16)]
                inc = plsc.cumsum(mi)
                pos = cntv + inc - mi
                plsc.store_scatter(psrc, [pos], s, mask=m)
                plsc.store_scatter(pdl, [pos], d - lo, mask=m)
                return cntv + plsc.cummax(lax.rev(inc, (0,)))
            return lax.fori_loop(0, CH // 16, vec_body, cntv)

        def process(cntv, infl, gsk):
            tot = jnp.max(cntv)
            nb = tot >> 4

            def fb(i, u):
                infl, gsk = u
                base = i * 16
                return flush_any(pdl[pl.ds(base, 16)], psrc[pl.ds(base, 16)],
                                 infl, gsk)
            infl, gsk = lax.fori_loop(0, nb, fb, (infl, gsk))
            rem = tot - nb * 16
            ls = psrc[pl.ds(nb * 16, 16)]
            ld_ = pdl[pl.ds(nb * 16, 16)]
            remm = iota < rem
            plsc.store_scatter(psrc, [iota], ls, mask=remm)
            plsc.store_scatter(pdl, [iota], ld_, mask=remm)
            return jnp.broadcast_to(rem, (16,)), infl, gsk

        def issue(ci, e0v, e1v, s):
            pltpu.async_copy(e0h.at[pl.ds(ci * CH, CH)], e0v, s)
            pltpu.async_copy(e1h.at[pl.ds(ci * CH, CH)], e1v, s)

        def drain(ci, e0v, e1v, s):
            pltpu.make_async_copy(e0h.at[pl.ds(ci * CH, CH)], e0v, s).wait()
            pltpu.make_async_copy(e1h.at[pl.ds(ci * CH, CH)], e1v, s).wait()

        def pass_body(p, _):
            lo = (p * 32 + wid) * R

            def initbody(i, _):
                for k in range(8):
                    acc[i, pl.ds(k * 16, 16)] = jnp.full((16,), SENT,
                                                         jnp.float32)
                return 0
            lax.fori_loop(0, R, initbody, 0)
            zeros = jnp.zeros((16,), jnp.int32)
            psrc[pl.ds(0, 16)] = zeros
            psrc[pl.ds(16, 16)] = zeros
            pdl[pl.ds(0, 16)] = zeros
            pdl[pl.ds(16, 16)] = zeros

            issue(jnp.int32(0), e0a, e1a, sema)
            issue(jnp.int32(1), e0b, e1b, semb)
            t = (jnp.zeros((16,), jnp.int32), jnp.int32(0),
                 jnp.zeros((16,), jnp.int32))

            def pair_body(pi, t):
                cntv, infl, gsk = t
                ca = pi * 2
                drain(ca, e0a, e1a, sema)
                cntv = scan_chunk(e0a, e1a, lo, cntv)
                issue(ca + 2, e0a, e1a, sema)
                cntv, infl, gsk = process(cntv, infl, gsk)
                drain(ca + 1, e0b, e1b, semb)
                cntv = scan_chunk(e0b, e1b, lo, cntv)
                issue(ca + 3, e0b, e1b, semb)
                return process(cntv, infl, gsk)

            cntv, infl, gsk = lax.fori_loop(0, nchunks // 2, pair_body, t)
            drain(jnp.int32(nchunks), e0a, e1a, sema)
            drain(jnp.int32(nchunks + 1), e0b, e1b, semb)
            rem = jnp.max(cntv)

            def finalflush(u):
                infl, gsk = u
                dlv = jnp.where(iota < rem, pdl[pl.ds(0, 16)], big)
                return flush_any(dlv, psrc[pl.ds(0, 16)], infl, gsk)
            infl, gsk = lax.cond(rem > 0, finalflush, lambda u: u,
                                 (infl, gsk))

            @pl.when(infl > 0)
            def _():
                gather_wait()
                do_rmw(gsk)

            pltpu.sync_copy(acc, out.at[pl.ds(lo, R)])
            return 0

        lax.fori_loop(0, NPASS, pass_body, 0)

    return segmin


def _prep_edges(e_src, e_dst):
    e = e_src.shape[0]
    ep = ((e + 2 * CH - 1) // (2 * CH)) * (2 * CH)
    s = jnp.pad(e_src, (0, ep + 2 * CH - e))
    d = jnp.pad(e_dst, (0, ep + 2 * CH - e), constant_values=1 << 20)
    return s, d, ep


def kernel(vertices, edges, faces, edge_to_vertex, face_to_edge, face_to_face,
           W_v, b_v, W_e, b_e, W_f, b_f, W_v2e, b_v2e, W_e2f, b_e2f,
           W_l0, b_l0, W_l1, b_l1, W_l2, b_l2):
    def padfeat(x, w):
        xp = jnp.pad(x, ((0, NP - N), (0, 128 - x.shape[1])))
        wp = jnp.pad(w, ((0, 128 - w.shape[0]), (0, 0)))
        return xp, wp

    xv_p, wv_p = padfeat(vertices, W_v)
    xe_p, we_p = padfeat(edges, W_e)
    xf_p, wf_p = padfeat(faces, W_f)
    enc = _encode(jnp.stack([xv_p, xe_p, xf_p]),
                  jnp.stack([wv_p, we_p, wf_p]),
                  jnp.stack([b_v, b_e, b_f])[:, None, :])
    x_v, x_e, x_f = enc[0], enc[1], enc[2]

    ev_s, ev_d, ev_n = _prep_edges(edge_to_vertex[1], edge_to_vertex[0])
    fe_s, fe_d, fe_n = _prep_edges(face_to_edge[1], face_to_edge[0])
    ff_s, ff_d, ff_n = _prep_edges(face_to_face[0], face_to_face[1])

    mn = _make_segmin(ev_n)(x_v, ev_s, ev_d)
    x_e = _dense(x_e, mn, W_v2e, b_v2e)
    mn = _make_segmin(fe_n)(x_e, fe_s, fe_d)
    x_f = _dense(x_f, mn, W_e2f, b_e2f)
    segff = _make_segmin(ff_n)
    for w, b in ((W_l0, b_l0), (W_l1, b_l1), (W_l2, b_l2)):
        mn = segff(x_f, ff_s, ff_d)
        x_f = _dense(x_f, mn, w, b)
    return x_f[:N]
